# baseline (speedup 1.0000x reference)
import functools

import jax
import jax.numpy as jnp
from jax import lax
from jax.experimental import pallas as pl
from jax.experimental.pallas import tpu as pltpu
from jax.experimental.pallas import tpu_sc as plsc

VOCAB = 1000000
EMBED = 128
HALF = EMBED // 2
SEQ = 200
BATCH = 4096


_TBLK = 4000


def _transform_body(tab_ref, w_ref, b_ref, out_ref):
    x = tab_ref[...]
    y = lax.dot_general(
        x, w_ref[...], (((1,), (1,)), ((), ())),
        preferred_element_type=jnp.float32,
    ) + b_ref[...]
    v = lax.bitcast_convert_type(
        y.astype(jnp.bfloat16).astype(jnp.float32), jnp.int32)
    m = v ^ ((v >> 31) & jnp.int32(0x7FFFFFFF))
    lo = lax.shift_right_logical(m[:, :HALF], 16)
    hi = m[:, HALF:] & jnp.int32(-65536)
    out_ref[...] = lo | hi


def _transform_table(table, W, b):
    return pl.pallas_call(
        _transform_body,
        grid=(VOCAB // _TBLK,),
        in_specs=[
            pl.BlockSpec((_TBLK, EMBED), lambda i: (i, 0)),
            pl.BlockSpec((EMBED, EMBED), lambda i: (0, 0)),
            pl.BlockSpec((1, EMBED), lambda i: (0, 0)),
        ],
        out_specs=pl.BlockSpec((_TBLK, HALF), lambda i: (i, 0)),
        out_shape=jax.ShapeDtypeStruct((VOCAB, HALF), jnp.int32),
    )(table, W, b.reshape(1, EMBED))


_NC = 2
_NS = 16
_NW = _NC * _NS
_COLS_PER_W = BATCH // _NW
_TOK_PER_W = _COLS_PER_W * SEQ
_CH0 = 104
_CH1 = SEQ - _CH0
_NVR = HALF // 16


@functools.partial(
    pl.kernel,
    out_type=jax.ShapeDtypeStruct((BATCH, EMBED), jnp.int32),
    mesh=plsc.VectorSubcoreMesh(core_axis_name="c", subcore_axis_name="s"),
    compiler_params=pltpu.CompilerParams(use_tc_tiling_on_sc=False),
    scratch_types=[
        pltpu.VMEM((_TOK_PER_W,), jnp.int32),
        pltpu.VMEM((SEQ, HALF), jnp.int32),
        pltpu.VMEM((SEQ, HALF), jnp.int32),
        pltpu.VMEM((_COLS_PER_W, EMBED), jnp.int32),
        pltpu.SemaphoreType.DMA,
        pltpu.SemaphoreType.DMA,
    ],
)
def _gather_max(tp_hbm, idx_hbm, out_hbm, idx_v, buf_a, buf_b, acc_v,
                sem_a, sem_b):
    wid = lax.axis_index("s") * _NC + lax.axis_index("c")

    tok_base = pl.multiple_of(wid * _TOK_PER_W, 8)
    pltpu.sync_copy(idx_hbm.at[pl.ds(tok_base, _TOK_PER_W)], idx_v)

    def issue(c, buf, sem):
        base = pl.multiple_of(c * SEQ, 8)
        pltpu.async_copy(
            tp_hbm.at[idx_v.at[pl.ds(base, _CH0)]],
            buf.at[pl.ds(0, _CH0)], sem)
        pltpu.async_copy(
            tp_hbm.at[idx_v.at[pl.ds(base + _CH0, _CH1)]],
            buf.at[pl.ds(_CH0, _CH1)], sem)

    def drain(buf, sem):
        pltpu.make_async_copy(tp_hbm.at[pl.ds(0, SEQ)], buf, sem).wait()

    def reduce_store(c, buf):
        def red(s, acc):
            new = []
            for k in range(_NVR):
                u = buf[s, pl.ds(16 * k, 16)]
                new.append(jnp.maximum(acc[2 * k], u << 16))
                new.append(jnp.maximum(acc[2 * k + 1], u))
            return tuple(new)

        neg = jnp.full((16,), jnp.int32(-2**31), jnp.int32)
        acc = lax.fori_loop(0, SEQ, red, tuple(neg for _ in range(2 * _NVR)))
        for k in range(_NVR):
            acc_v[c, pl.ds(16 * k, 16)] = acc[2 * k]
            acc_v[c, pl.ds(HALF + 16 * k, 16)] = acc[2 * k + 1]

    issue(0, buf_a, sem_a)

    def body(i, carry):
        c0 = 2 * i
        issue(c0 + 1, buf_b, sem_b)
        drain(buf_a, sem_a)
        reduce_store(c0, buf_a)

        @pl.when(i < _COLS_PER_W // 2 - 1)
        def _():
            issue(c0 + 2, buf_a, sem_a)

        drain(buf_b, sem_b)
        reduce_store(c0 + 1, buf_b)
        return carry

    lax.fori_loop(0, _COLS_PER_W // 2, body, 0)

    col_base = pl.multiple_of(wid * _COLS_PER_W, 8)
    pltpu.sync_copy(acc_v, out_hbm.at[pl.ds(col_base, _COLS_PER_W)])


def kernel(inputs, table, W, b):
    tp = _transform_table(table, W, b)
    idx = jnp.asarray(inputs, jnp.int32).T.reshape(-1)
    packed = _gather_max(tp, idx)
    m = (packed >> 16).astype(jnp.int16)
    bits = m ^ ((m >> 15) & jnp.int16(0x7FFF))
    return lax.bitcast_convert_type(bits, jnp.bfloat16).astype(jnp.float32)


# speedup vs baseline: 1.9928x; 1.9928x over previous
import functools

import jax
import jax.numpy as jnp
from jax import lax
from jax.experimental import pallas as pl
from jax.experimental.pallas import tpu as pltpu
from jax.experimental.pallas import tpu_sc as plsc

VOCAB = 1000000
EMBED = 128
HALF = EMBED // 2
SEQ = 200
BATCH = 4096


_TBLK = 4000


def _pack_rows(y):
    v = lax.bitcast_convert_type(
        y.astype(jnp.bfloat16).astype(jnp.float32), jnp.int32)
    m = v ^ ((v >> 31) & jnp.int32(0x7FFFFFFF))
    lo = lax.shift_right_logical(m[:, :HALF], 16)
    hi = m[:, HALF:] & jnp.int32(-65536)
    return lo | hi


def _transform_body(tab_a_ref, tab_b_ref, w_ref, b_ref, out_ref):
    w = w_ref[...]
    bias = b_ref[...]

    def tf(x):
        return lax.dot_general(
            x, w, (((1,), (1,)), ((), ())),
            preferred_element_type=jnp.float32,
        ) + bias

    out_ref[...] = jnp.concatenate(
        [_pack_rows(tf(tab_a_ref[...])), _pack_rows(tf(tab_b_ref[...]))],
        axis=1)


def _transform_table(table, W, b):
    nblk = VOCAB // 2 // _TBLK
    return pl.pallas_call(
        _transform_body,
        grid=(nblk,),
        in_specs=[
            pl.BlockSpec((_TBLK, EMBED), lambda i: (i, 0)),
            pl.BlockSpec((_TBLK, EMBED), lambda i, n=nblk: (i + n, 0)),
            pl.BlockSpec((EMBED, EMBED), lambda i: (0, 0)),
            pl.BlockSpec((1, EMBED), lambda i: (0, 0)),
        ],
        out_specs=pl.BlockSpec((_TBLK, EMBED), lambda i: (i, 0)),
        out_shape=jax.ShapeDtypeStruct((VOCAB // 2, EMBED), jnp.int32),
    )(table, table, W, b.reshape(1, EMBED))


_NC = 2
_NS = 16
_NW = _NC * _NS
_COLS_PER_W = BATCH // _NW
_TOK_PER_W = _COLS_PER_W * SEQ
_CH0 = 104
_CH1 = SEQ - _CH0
_NVR = HALF // 16


@functools.partial(
    pl.kernel,
    out_type=jax.ShapeDtypeStruct((BATCH, EMBED), jnp.int32),
    mesh=plsc.VectorSubcoreMesh(core_axis_name="c", subcore_axis_name="s"),
    compiler_params=pltpu.CompilerParams(use_tc_tiling_on_sc=False),
    scratch_types=[
        pltpu.VMEM((_TOK_PER_W,), jnp.int32),
        pltpu.VMEM((SEQ, HALF), jnp.int32),
        pltpu.VMEM((SEQ, HALF), jnp.int32),
        pltpu.VMEM((_COLS_PER_W, EMBED), jnp.int32),
        pltpu.SemaphoreType.DMA,
        pltpu.SemaphoreType.DMA,
    ],
)
def _gather_max(tp_hbm, idx_hbm, out_hbm, idx_v, buf_a, buf_b, acc_v,
                sem_a, sem_b):
    wid = lax.axis_index("s") * _NC + lax.axis_index("c")

    tok_base = pl.multiple_of(wid * _TOK_PER_W, 8)
    pltpu.sync_copy(idx_hbm.at[pl.ds(tok_base, _TOK_PER_W)], idx_v)

    def issue(c, buf, sem):
        base = pl.multiple_of(c * SEQ, 8)
        pltpu.async_copy(
            tp_hbm.at[idx_v.at[pl.ds(base, _CH0)]],
            buf.at[pl.ds(0, _CH0)], sem)
        pltpu.async_copy(
            tp_hbm.at[idx_v.at[pl.ds(base + _CH0, _CH1)]],
            buf.at[pl.ds(_CH0, _CH1)], sem)

    def drain(buf, sem):
        pltpu.make_async_copy(tp_hbm.at[pl.ds(0, SEQ)], buf, sem).wait()

    def reduce_store(c, buf):
        def red(s, acc):
            new = []
            for k in range(_NVR):
                u = buf[s, pl.ds(16 * k, 16)]
                new.append(jnp.maximum(acc[2 * k], u << 16))
                new.append(jnp.maximum(acc[2 * k + 1], u))
            return tuple(new)

        neg = jnp.full((16,), jnp.int32(-2**31), jnp.int32)
        acc = lax.fori_loop(0, SEQ, red, tuple(neg for _ in range(2 * _NVR)))
        for k in range(_NVR):
            acc_v[c, pl.ds(16 * k, 16)] = acc[2 * k]
            acc_v[c, pl.ds(HALF + 16 * k, 16)] = acc[2 * k + 1]

    issue(0, buf_a, sem_a)

    def body(i, carry):
        c0 = 2 * i
        issue(c0 + 1, buf_b, sem_b)
        drain(buf_a, sem_a)
        reduce_store(c0, buf_a)

        @pl.when(i < _COLS_PER_W // 2 - 1)
        def _():
            issue(c0 + 2, buf_a, sem_a)

        drain(buf_b, sem_b)
        reduce_store(c0 + 1, buf_b)
        return carry

    lax.fori_loop(0, _COLS_PER_W // 2, body, 0)

    col_base = pl.multiple_of(wid * _COLS_PER_W, 8)
    pltpu.sync_copy(acc_v, out_hbm.at[pl.ds(col_base, _COLS_PER_W)])


def kernel(inputs, table, W, b):
    tp = _transform_table(table, W, b).reshape(VOCAB, HALF)
    v = jnp.asarray(inputs, jnp.int32).T.reshape(-1)
    idx = jnp.where(v < VOCAB // 2, v * 2, v * 2 - (VOCAB - 1))
    packed = _gather_max(tp, idx)
    m = (packed >> 16).astype(jnp.int16)
    bits = m ^ ((m >> 15) & jnp.int16(0x7FFF))
    return lax.bitcast_convert_type(bits, jnp.bfloat16).astype(jnp.float32)


# speedup vs baseline: 2.2466x; 1.1274x over previous
import functools

import jax
import jax.numpy as jnp
from jax import lax
from jax.experimental import pallas as pl
from jax.experimental.pallas import tpu as pltpu
from jax.experimental.pallas import tpu_sc as plsc

VOCAB = 1000000
EMBED = 128
HALF = EMBED // 2
SEQ = 200
BATCH = 4096


_TBLK = 10000


def _pack_rows(y):
    v = lax.bitcast_convert_type(
        y.astype(jnp.bfloat16).astype(jnp.float32), jnp.int32)
    m = v ^ ((v >> 31) & jnp.int32(0x7FFFFFFF))
    lo = lax.shift_right_logical(m[:, :HALF], 16)
    hi = m[:, HALF:] & jnp.int32(-65536)
    return lo | hi


def _transform_body(tab_a_ref, tab_b_ref, w_ref, b_ref, out_ref):
    w = w_ref[...]
    bias = b_ref[...]

    def tf(x):
        return lax.dot_general(
            x, w, (((1,), (1,)), ((), ())),
            preferred_element_type=jnp.float32,
        ) + bias

    out_ref[...] = jnp.concatenate(
        [_pack_rows(tf(tab_a_ref[...])), _pack_rows(tf(tab_b_ref[...]))],
        axis=1)


def _transform_table(table, W, b):
    nblk = VOCAB // 2 // _TBLK
    return pl.pallas_call(
        _transform_body,
        grid=(nblk,),
        in_specs=[
            pl.BlockSpec((_TBLK, EMBED), lambda i: (i, 0)),
            pl.BlockSpec((_TBLK, EMBED), lambda i, n=nblk: (i + n, 0)),
            pl.BlockSpec((EMBED, EMBED), lambda i: (0, 0)),
            pl.BlockSpec((1, EMBED), lambda i: (0, 0)),
        ],
        out_specs=pl.BlockSpec((_TBLK, EMBED), lambda i: (i, 0)),
        out_shape=jax.ShapeDtypeStruct((VOCAB // 2, EMBED), jnp.int32),
    )(table, table, W, b.reshape(1, EMBED))


_NC = 2
_NS = 16
_NW = _NC * _NS
_COLS_PER_W = BATCH // _NW
_TOK_PER_W = _COLS_PER_W * SEQ
_CH0 = 104
_CH1 = SEQ - _CH0
_NVR = HALF // 16


@functools.partial(
    pl.kernel,
    out_type=jax.ShapeDtypeStruct((BATCH, EMBED), jnp.int32),
    mesh=plsc.VectorSubcoreMesh(core_axis_name="c", subcore_axis_name="s"),
    compiler_params=pltpu.CompilerParams(use_tc_tiling_on_sc=False),
    scratch_types=[
        pltpu.VMEM((_TOK_PER_W,), jnp.int32),
        pltpu.VMEM((SEQ, HALF), jnp.int32),
        pltpu.VMEM((SEQ, HALF), jnp.int32),
        pltpu.VMEM((_COLS_PER_W, EMBED), jnp.int32),
        pltpu.SemaphoreType.DMA,
        pltpu.SemaphoreType.DMA,
    ],
)
def _gather_max(tp_hbm, idx_hbm, out_hbm, idx_v, buf_a, buf_b, acc_v,
                sem_a, sem_b):
    wid = lax.axis_index("s") * _NC + lax.axis_index("c")

    tok_base = pl.multiple_of(wid * _TOK_PER_W, 8)
    pltpu.sync_copy(idx_hbm.at[pl.ds(tok_base, _TOK_PER_W)], idx_v)

    def issue(c, buf, sem):
        base = pl.multiple_of(c * SEQ, 8)
        pltpu.async_copy(
            tp_hbm.at[idx_v.at[pl.ds(base, _CH0)]],
            buf.at[pl.ds(0, _CH0)], sem)
        pltpu.async_copy(
            tp_hbm.at[idx_v.at[pl.ds(base + _CH0, _CH1)]],
            buf.at[pl.ds(_CH0, _CH1)], sem)

    def drain(buf, sem):
        pltpu.make_async_copy(tp_hbm.at[pl.ds(0, SEQ)], buf, sem).wait()

    def reduce_store(c, buf):
        def red(s, acc):
            new = []
            for k in range(_NVR):
                u = buf[s, pl.ds(16 * k, 16)]
                new.append(jnp.maximum(acc[2 * k], u << 16))
                new.append(jnp.maximum(acc[2 * k + 1], u))
            return tuple(new)

        neg = jnp.full((16,), jnp.int32(-2**31), jnp.int32)
        acc = lax.fori_loop(0, SEQ, red, tuple(neg for _ in range(2 * _NVR)))
        for k in range(_NVR):
            acc_v[c, pl.ds(16 * k, 16)] = acc[2 * k]
            acc_v[c, pl.ds(HALF + 16 * k, 16)] = acc[2 * k + 1]

    issue(0, buf_a, sem_a)

    def body(i, carry):
        c0 = 2 * i
        issue(c0 + 1, buf_b, sem_b)
        drain(buf_a, sem_a)
        reduce_store(c0, buf_a)

        @pl.when(i < _COLS_PER_W // 2 - 1)
        def _():
            issue(c0 + 2, buf_a, sem_a)

        drain(buf_b, sem_b)
        reduce_store(c0 + 1, buf_b)
        return carry

    lax.fori_loop(0, _COLS_PER_W // 2, body, 0)

    col_base = pl.multiple_of(wid * _COLS_PER_W, 8)
    pltpu.sync_copy(acc_v, out_hbm.at[pl.ds(col_base, _COLS_PER_W)])


def kernel(inputs, table, W, b):
    tp = _transform_table(table, W, b).reshape(VOCAB, HALF)
    v = jnp.asarray(inputs, jnp.int32).T.reshape(-1)
    idx = jnp.where(v < VOCAB // 2, v * 2, v * 2 - (VOCAB - 1))
    packed = _gather_max(tp, idx)
    m = (packed >> 16).astype(jnp.int16)
    bits = m ^ ((m >> 15) & jnp.int16(0x7FFF))
    return lax.bitcast_convert_type(bits, jnp.bfloat16).astype(jnp.float32)


# speedup vs baseline: 2.3144x; 1.0302x over previous
import functools

import jax
import jax.numpy as jnp
from jax import lax
from jax.experimental import pallas as pl
from jax.experimental.pallas import tpu as pltpu
from jax.experimental.pallas import tpu_sc as plsc

VOCAB = 1000000
EMBED = 128
HALF = EMBED // 2
SEQ = 200
BATCH = 4096


_TBLK = 10000


def _pack_rows(y):
    v = lax.bitcast_convert_type(
        y.astype(jnp.bfloat16).astype(jnp.float32), jnp.int32)
    m = v ^ ((v >> 31) & jnp.int32(0x7FFFFFFF))
    lo = lax.shift_right_logical(m[:, :HALF], 16)
    hi = m[:, HALF:] & jnp.int32(-65536)
    return lo | hi


def _transform_body(tab_a_ref, tab_b_ref, w_ref, b_ref, out_ref):
    w = w_ref[...]
    bias = b_ref[...]

    def tf(x):
        return lax.dot_general(
            x, w, (((1,), (1,)), ((), ())),
            preferred_element_type=jnp.float32,
        ) + bias

    out_ref[...] = jnp.concatenate(
        [_pack_rows(tf(tab_a_ref[...])), _pack_rows(tf(tab_b_ref[...]))],
        axis=1)


def _transform_table(table, W, b):
    nblk = VOCAB // 2 // _TBLK
    return pl.pallas_call(
        _transform_body,
        grid=(nblk,),
        in_specs=[
            pl.BlockSpec((_TBLK, EMBED), lambda i: (i, 0)),
            pl.BlockSpec((_TBLK, EMBED), lambda i, n=nblk: (i + n, 0)),
            pl.BlockSpec((EMBED, EMBED), lambda i: (0, 0)),
            pl.BlockSpec((1, EMBED), lambda i: (0, 0)),
        ],
        out_specs=pl.BlockSpec((_TBLK, EMBED), lambda i: (i, 0)),
        out_shape=jax.ShapeDtypeStruct((VOCAB // 2, EMBED), jnp.int32),
    )(table, table, W, b.reshape(1, EMBED))


_NC = 2
_NS = 16
_NW = _NC * _NS
_COLS_PER_W = BATCH // _NW
_TOK_PER_W = _COLS_PER_W * SEQ
_CH0 = 104
_CH1 = SEQ - _CH0
_NVR = HALF // 16


@functools.partial(
    pl.kernel,
    out_type=jax.ShapeDtypeStruct((BATCH, EMBED), jnp.int32),
    mesh=plsc.VectorSubcoreMesh(core_axis_name="c", subcore_axis_name="s"),
    compiler_params=pltpu.CompilerParams(use_tc_tiling_on_sc=False),
    scratch_types=[
        pltpu.VMEM((_TOK_PER_W,), jnp.int32),
        pltpu.VMEM((SEQ, HALF), jnp.int32),
        pltpu.VMEM((SEQ, HALF), jnp.int32),
        pltpu.VMEM((_COLS_PER_W, EMBED), jnp.int32),
        pltpu.SemaphoreType.DMA,
        pltpu.SemaphoreType.DMA,
    ],
)
def _gather_max(tp_hbm, idx_hbm, out_hbm, idx_v, buf_a, buf_b, acc_v,
                sem_a, sem_b):
    wid = lax.axis_index("s") * _NC + lax.axis_index("c")

    tok_base = pl.multiple_of(wid * _TOK_PER_W, 8)
    pltpu.sync_copy(idx_hbm.at[pl.ds(tok_base, _TOK_PER_W)], idx_v)

    def issue(c, buf, sem):
        base = pl.multiple_of(c * SEQ, 8)
        pltpu.async_copy(
            tp_hbm.at[idx_v.at[pl.ds(base, _CH0)]],
            buf.at[pl.ds(0, _CH0)], sem)
        pltpu.async_copy(
            tp_hbm.at[idx_v.at[pl.ds(base + _CH0, _CH1)]],
            buf.at[pl.ds(_CH0, _CH1)], sem)

    def drain(buf, sem):
        pltpu.make_async_copy(tp_hbm.at[pl.ds(0, SEQ)], buf, sem).wait()

    def reduce_store(c, buf):
        def red(s, acc):
            new = []
            for k in range(_NVR):
                u = buf[s, pl.ds(16 * k, 16)]
                new.append(jnp.maximum(acc[2 * k], u << 16))
                new.append(jnp.maximum(acc[2 * k + 1], u))
            return tuple(new)

        neg = jnp.full((16,), jnp.int32(-2**31), jnp.int32)
        acc = lax.fori_loop(0, SEQ, red, tuple(neg for _ in range(2 * _NVR)))
        for k in range(_NVR):
            acc_v[c, pl.ds(16 * k, 16)] = acc[2 * k]
            acc_v[c, pl.ds(HALF + 16 * k, 16)] = acc[2 * k + 1]

    issue(0, buf_a, sem_a)

    def body(i, carry):
        c0 = 2 * i
        issue(c0 + 1, buf_b, sem_b)
        drain(buf_a, sem_a)
        reduce_store(c0, buf_a)

        @pl.when(i < _COLS_PER_W // 2 - 1)
        def _():
            issue(c0 + 2, buf_a, sem_a)

        drain(buf_b, sem_b)
        reduce_store(c0 + 1, buf_b)
        return carry

    lax.fori_loop(0, _COLS_PER_W // 2, body, 0)

    col_base = pl.multiple_of(wid * _COLS_PER_W, 8)
    pltpu.sync_copy(acc_v, out_hbm.at[pl.ds(col_base, _COLS_PER_W)])


def kernel(inputs, table, W, b):
    tp = _transform_table(table, W, b).reshape(VOCAB, HALF)
    v = jnp.asarray(inputs, jnp.int32)
    idx = jnp.where(v < VOCAB // 2, v * 2, v * 2 - (VOCAB - 1)).T.reshape(-1)
    packed = _gather_max(tp, idx)
    m = (packed >> 16).astype(jnp.int16)
    bits = m ^ ((m >> 15) & jnp.int16(0x7FFF))
    return lax.bitcast_convert_type(bits, jnp.bfloat16).astype(jnp.float32)


# speedup vs baseline: 2.3230x; 1.0037x over previous
import functools

import jax
import jax.numpy as jnp
from jax import lax
from jax.experimental import pallas as pl
from jax.experimental.pallas import tpu as pltpu
from jax.experimental.pallas import tpu_sc as plsc

VOCAB = 1000000
EMBED = 128
HALF = EMBED // 2
SEQ = 200
BATCH = 4096


_TBLK = 10000


def _pack_rows(y):
    v = lax.bitcast_convert_type(
        y.astype(jnp.bfloat16).astype(jnp.float32), jnp.int32)
    m = v ^ ((v >> 31) & jnp.int32(0x7FFFFFFF))
    lo = lax.shift_right_logical(m[:, :HALF], 16)
    hi = m[:, HALF:] & jnp.int32(-65536)
    return lo | hi


def _transform_body(tab_a_ref, tab_b_ref, w_ref, b_ref, out_ref):
    w = w_ref[...]
    bias = b_ref[...]

    def tf(x):
        return lax.dot_general(
            x, w, (((1,), (1,)), ((), ())),
            preferred_element_type=jnp.float32,
        ) + bias

    out_ref[...] = jnp.concatenate(
        [_pack_rows(tf(tab_a_ref[...])), _pack_rows(tf(tab_b_ref[...]))],
        axis=1)


def _transform_table(table, W, b):
    nblk = VOCAB // 2 // _TBLK
    return pl.pallas_call(
        _transform_body,
        grid=(nblk,),
        in_specs=[
            pl.BlockSpec((_TBLK, EMBED), lambda i: (i, 0)),
            pl.BlockSpec((_TBLK, EMBED), lambda i, n=nblk: (i + n, 0)),
            pl.BlockSpec((EMBED, EMBED), lambda i: (0, 0)),
            pl.BlockSpec((1, EMBED), lambda i: (0, 0)),
        ],
        out_specs=pl.BlockSpec((_TBLK, EMBED), lambda i: (i, 0)),
        out_shape=jax.ShapeDtypeStruct((VOCAB // 2, EMBED), jnp.int32),
    )(table, table, W, b.reshape(1, EMBED))


_NC = 2
_NS = 16
_NW = _NC * _NS
_COLS_PER_W = BATCH // _NW
_TOK_PER_W = _COLS_PER_W * SEQ
_CH0 = 104
_CH1 = SEQ - _CH0
_NVR = HALF // 16
_RUNROLL = 4


@functools.partial(
    pl.kernel,
    out_type=jax.ShapeDtypeStruct((BATCH, EMBED), jnp.int32),
    mesh=plsc.VectorSubcoreMesh(core_axis_name="c", subcore_axis_name="s"),
    compiler_params=pltpu.CompilerParams(use_tc_tiling_on_sc=False),
    scratch_types=[
        pltpu.VMEM((_TOK_PER_W,), jnp.int32),
        pltpu.VMEM((SEQ, HALF), jnp.int32),
        pltpu.VMEM((SEQ, HALF), jnp.int32),
        pltpu.VMEM((_COLS_PER_W, EMBED), jnp.int32),
        pltpu.SemaphoreType.DMA,
        pltpu.SemaphoreType.DMA,
    ],
)
def _gather_max(tp_hbm, idx_hbm, out_hbm, idx_v, buf_a, buf_b, acc_v,
                sem_a, sem_b):
    wid = lax.axis_index("s") * _NC + lax.axis_index("c")

    tok_base = pl.multiple_of(wid * _TOK_PER_W, 8)
    pltpu.sync_copy(idx_hbm.at[pl.ds(tok_base, _TOK_PER_W)], idx_v)

    def issue(c, buf, sem):
        base = pl.multiple_of(c * SEQ, 8)
        pltpu.async_copy(
            tp_hbm.at[idx_v.at[pl.ds(base, _CH0)]],
            buf.at[pl.ds(0, _CH0)], sem)
        pltpu.async_copy(
            tp_hbm.at[idx_v.at[pl.ds(base + _CH0, _CH1)]],
            buf.at[pl.ds(_CH0, _CH1)], sem)

    def drain(buf, sem):
        pltpu.make_async_copy(tp_hbm.at[pl.ds(0, SEQ)], buf, sem).wait()

    def reduce_store(c, buf):
        def red(s, acc):
            new = list(acc)
            for r in range(_RUNROLL):
                for k in range(_NVR):
                    u = buf[s * _RUNROLL + r, pl.ds(16 * k, 16)]
                    new[2 * k] = jnp.maximum(new[2 * k], u << 16)
                    new[2 * k + 1] = jnp.maximum(new[2 * k + 1], u)
            return tuple(new)

        neg = jnp.full((16,), jnp.int32(-2**31), jnp.int32)
        acc = lax.fori_loop(0, SEQ // _RUNROLL, red,
                            tuple(neg for _ in range(2 * _NVR)))
        for k in range(_NVR):
            acc_v[c, pl.ds(16 * k, 16)] = acc[2 * k]
            acc_v[c, pl.ds(HALF + 16 * k, 16)] = acc[2 * k + 1]

    issue(0, buf_a, sem_a)

    def body(i, carry):
        c0 = 2 * i
        issue(c0 + 1, buf_b, sem_b)
        drain(buf_a, sem_a)
        reduce_store(c0, buf_a)

        @pl.when(i < _COLS_PER_W // 2 - 1)
        def _():
            issue(c0 + 2, buf_a, sem_a)

        drain(buf_b, sem_b)
        reduce_store(c0 + 1, buf_b)
        return carry

    lax.fori_loop(0, _COLS_PER_W // 2, body, 0)

    col_base = pl.multiple_of(wid * _COLS_PER_W, 8)
    pltpu.sync_copy(acc_v, out_hbm.at[pl.ds(col_base, _COLS_PER_W)])


def kernel(inputs, table, W, b):
    tp = _transform_table(table, W, b).reshape(VOCAB, HALF)
    v = jnp.asarray(inputs, jnp.int32)
    idx = jnp.where(v < VOCAB // 2, v * 2, v * 2 - (VOCAB - 1)).T.reshape(-1)
    packed = _gather_max(tp, idx)
    m = (packed >> 16).astype(jnp.int16)
    bits = m ^ ((m >> 15) & jnp.int16(0x7FFF))
    return lax.bitcast_convert_type(bits, jnp.bfloat16).astype(jnp.float32)
